# Initial kernel scaffold; baseline (speedup 1.0000x reference)
import jax
import jax.numpy as jnp
from jax.experimental import pallas as pl


def kernel(modal1_inputs, modal2_inputs, targets, centers_param):
    raise NotImplementedError("write your pallas kernel here")


# speedup vs baseline: 2.9342x; 2.9342x over previous
import functools

import jax
import jax.numpy as jnp
from jax import lax
from jax.experimental import pallas as pl
from jax.experimental.pallas import tpu as pltpu
from jax.experimental.pallas import tpu_sc as plsc

_B = 4096
_D = 512
_C = 1000
_L = 16
_NCORE = 2
_NSUB = 16
_NW = _NCORE * _NSUB
_FPW = _D // _NW
_CHUNK = 512
_NCHUNK = _B // _CHUNK
_GROUPS = _CHUNK // _L
_CPAD = 1008


_GATHER_DNUMS = lax.GatherDimensionNumbers(
    offset_dims=(), collapsed_slice_dims=(0,), start_index_map=(0,))


def _bcast_lane(vec, k):
    idx = jnp.full((_L, 1), k, jnp.int32)
    return lax.gather(vec, idx, _GATHER_DNUMS, slice_sizes=(1,),
                      mode=lax.GatherScatterMode.PROMISE_IN_BOUNDS)


def _sc_body(m1_hbm, m2_hbm, tgt_hbm, cent_hbm, out_hbm,
             tgt_v, m1_v, m2_v, cent_v, s1_v, s2_v, cnt_v, res_v,
             tsem, csem, msem):
    cid = lax.axis_index("c")
    sid = lax.axis_index("s")
    wid = sid * _NCORE + cid
    f0 = wid * _FPW

    iota = lax.iota(jnp.int32, _L)
    lane0 = iota == 0
    ones = jnp.ones((_L,), jnp.float32)
    zeros = jnp.zeros((_L,), jnp.float32)

    def _modal_copies(c):
        p = c % 2
        sl = pl.ds(c * _CHUNK, _CHUNK)
        a = pltpu.make_async_copy(
            m1_hbm.at[sl, pl.ds(f0, _FPW)], m1_v.at[p], msem.at[2 * p])
        b = pltpu.make_async_copy(
            m2_hbm.at[sl, pl.ds(f0, _FPW)], m2_v.at[p], msem.at[2 * p + 1])
        return a, b

    tgt_cp = pltpu.make_async_copy(tgt_hbm, tgt_v, tsem)
    tgt_cp.start()
    cent_cp = pltpu.make_async_copy(cent_hbm.at[:, pl.ds(f0, _FPW)], cent_v, csem)
    cent_cp.start()
    a0, b0 = _modal_copies(0)
    a0.start()
    b0.start()

    def _zero_sums(i, carry):
        s1_v[pl.ds(i * _L, _L)] = zeros
        s2_v[pl.ds(i * _L, _L)] = zeros
        return carry
    lax.fori_loop(0, _C, _zero_sums, 0)

    def _zero_cnt(i, carry):
        cnt_v[pl.ds(i * _L, _L)] = zeros
        return carry
    lax.fori_loop(0, _CPAD // _L, _zero_cnt, 0)

    tgt_cp.wait()

    for c in range(_NCHUNK):
        a, b = _modal_copies(c)
        a.wait()
        b.wait()
        if c + 1 < _NCHUNK:
            na, nb = _modal_copies(c + 1)
            na.start()
            nb.start()
        p = c % 2

        def _group(g, carry, _c=c, _p=p):
            tvec = tgt_v[pl.ds(_c * _CHUNK + g * _L, _L)]
            for k in range(_L):
                tjb = _bcast_lane(tvec, k)
                j = g * _L + k
                fidx = tjb * _L + iota
                plsc.addupdate_scatter(s1_v, [fidx], m1_v[_p, j])
                plsc.addupdate_scatter(s2_v, [fidx], m2_v[_p, j])
                plsc.addupdate_scatter(cnt_v, [tjb], ones, mask=lane0)
            return carry
        lax.fori_loop(0, _GROUPS, _group, 0)

    cent_cp.wait()

    def _class_term(ci, cb):
        inv = 1.0 / jnp.maximum(cb, 1.0)
        ct = cent_v[ci]
        d1 = s1_v[pl.ds(ci * _L, _L)] * inv - ct
        a1 = jnp.abs(d1)
        h1 = jnp.where(a1 < 1.0, 0.5 * d1 * d1, a1 - 0.5)
        d2 = s2_v[pl.ds(ci * _L, _L)] * inv - ct
        a2 = jnp.abs(d2)
        h2 = jnp.where(a2 < 1.0, 0.5 * d2 * d2, a2 - 0.5)
        return cb * (h1 + h2)

    def _class_group(g, acc):
        cvec = cnt_v[pl.ds(g * _L, _L)]
        for k in range(_L):
            acc = acc + _class_term(g * _L + k, _bcast_lane(cvec, k))
        return acc
    acc = lax.fori_loop(0, _C // _L, _class_group, zeros)

    cvec = cnt_v[pl.ds((_C // _L) * _L, _L)]
    for k in range(_C % _L):
        acc = acc + _class_term((_C // _L) * _L + k, _bcast_lane(cvec, k))

    res_v[...] = acc
    pltpu.sync_copy(res_v, out_hbm.at[wid])


_sc_kernel = functools.partial(
    pl.kernel,
    out_type=jax.ShapeDtypeStruct((_NW, _L), jnp.float32),
    mesh=plsc.VectorSubcoreMesh(core_axis_name="c", subcore_axis_name="s"),
    compiler_params=pltpu.CompilerParams(
        use_tc_tiling_on_sc=False, needs_layout_passes=False),
    scratch_types=[
        pltpu.VMEM((_B,), jnp.int32),
        pltpu.VMEM((2, _CHUNK, _L), jnp.float32),
        pltpu.VMEM((2, _CHUNK, _L), jnp.float32),
        pltpu.VMEM((_C, _L), jnp.float32),
        pltpu.VMEM((_C * _L,), jnp.float32),
        pltpu.VMEM((_C * _L,), jnp.float32),
        pltpu.VMEM((_CPAD,), jnp.float32),
        pltpu.VMEM((_L,), jnp.float32),
        pltpu.SemaphoreType.DMA,
        pltpu.SemaphoreType.DMA,
        pltpu.SemaphoreType.DMA((4,)),
    ],
)(_sc_body)


def _tc_reduce_body(x_ref, o_ref):
    o_ref[...] = jnp.sum(x_ref[...]).reshape(1, 1) * (1.0 / (_B * _D))


def kernel(modal1_inputs, modal2_inputs, targets, centers_param):
    partials = _sc_kernel(modal1_inputs, modal2_inputs, targets, centers_param)
    out = pl.pallas_call(
        _tc_reduce_body,
        out_shape=jax.ShapeDtypeStruct((1, 1), jnp.float32),
    )(partials)
    return out[0, 0]


# speedup vs baseline: 3.3298x; 1.1348x over previous
import functools

import jax
import jax.numpy as jnp
from jax import lax
from jax.experimental import pallas as pl
from jax.experimental.pallas import tpu as pltpu
from jax.experimental.pallas import tpu_sc as plsc

_B = 4096
_D = 512
_C = 1000
_L = 16
_NCORE = 2
_NSUB = 16
_NW = _NCORE * _NSUB
_FPW = _D // _NW
_CHUNK = 512
_NCHUNK = _B // _CHUNK
_GROUPS = _CHUNK // _L
_CPAD = 1008


_GATHER_DNUMS = lax.GatherDimensionNumbers(
    offset_dims=(), collapsed_slice_dims=(0,), start_index_map=(0,))


def _bcast_lane(vec, k):
    idx = jnp.full((_L, 1), k, jnp.int32)
    return lax.gather(vec, idx, _GATHER_DNUMS, slice_sizes=(1,),
                      mode=lax.GatherScatterMode.PROMISE_IN_BOUNDS)


def _sc_body(m1_hbm, m2_hbm, tgt_hbm, cent_hbm, out_hbm,
             tgt_v, m1_v, m2_v, cent_v, s1_v, s2_v, cnt_v, inv_v, res_v,
             tsem, csem, msem):
    cid = lax.axis_index("c")
    sid = lax.axis_index("s")
    wid = sid * _NCORE + cid
    f0 = wid * _FPW

    iota = lax.iota(jnp.int32, _L)
    zeros = jnp.zeros((_L,), jnp.float32)

    def _modal_copies(c):
        p = c % 2
        sl = pl.ds(c * _CHUNK, _CHUNK)
        a = pltpu.make_async_copy(
            m1_hbm.at[sl, pl.ds(f0, _FPW)], m1_v.at[p], msem.at[2 * p])
        b = pltpu.make_async_copy(
            m2_hbm.at[sl, pl.ds(f0, _FPW)], m2_v.at[p], msem.at[2 * p + 1])
        return a, b

    tgt_cp = pltpu.make_async_copy(tgt_hbm, tgt_v, tsem)
    tgt_cp.start()
    cent_cp = pltpu.make_async_copy(cent_hbm.at[:, pl.ds(f0, _FPW)], cent_v, csem)
    cent_cp.start()
    a0, b0 = _modal_copies(0)
    a0.start()
    b0.start()

    @plsc.parallel_loop(0, _C, unroll=4)
    def _zero_sums(i):
        s1_v[pl.ds(i * _L, _L)] = zeros
        s2_v[pl.ds(i * _L, _L)] = zeros

    @plsc.parallel_loop(0, _CPAD // _L, unroll=4)
    def _zero_cnt(i):
        cnt_v[pl.ds(i * _L, _L)] = zeros

    tgt_cp.wait()

    for c in range(_NCHUNK):
        a, b = _modal_copies(c)
        a.wait()
        b.wait()
        if c + 1 < _NCHUNK:
            na, nb = _modal_copies(c + 1)
            na.start()
            nb.start()
        p = c % 2

        @plsc.parallel_loop(0, _GROUPS, unroll=2)
        def _group(g, _c=c, _p=p):
            tvec = tgt_v[pl.ds(_c * _CHUNK + g * _L, _L)]
            dup, last = plsc.scan_count(tvec)
            plsc.addupdate_scatter(
                cnt_v, [tvec], dup.astype(jnp.float32), mask=last)
            for k in range(_L):
                tjb = _bcast_lane(tvec, k)
                j = g * _L + k
                fidx = tjb * _L + iota
                plsc.addupdate_scatter(s1_v, [fidx], m1_v[_p, j])
                plsc.addupdate_scatter(s2_v, [fidx], m2_v[_p, j])

    cent_cp.wait()

    @plsc.parallel_loop(0, _CPAD // _L, unroll=4)
    def _inv_cnt(i):
        inv_v[pl.ds(i * _L, _L)] = 1.0 / jnp.maximum(cnt_v[pl.ds(i * _L, _L)], 1.0)

    def _class_term(ci, cb, inv):
        ct = cent_v[ci]
        d1 = s1_v[pl.ds(ci * _L, _L)] * inv - ct
        a1 = jnp.abs(d1)
        h1 = jnp.where(a1 < 1.0, 0.5 * d1 * d1, a1 - 0.5)
        d2 = s2_v[pl.ds(ci * _L, _L)] * inv - ct
        a2 = jnp.abs(d2)
        h2 = jnp.where(a2 < 1.0, 0.5 * d2 * d2, a2 - 0.5)
        return cb * (h1 + h2)

    accs0 = (zeros, zeros, zeros, zeros)

    @plsc.parallel_loop(0, _C // _L, carry=accs0)
    def _class_group(g, accs):
        cvec = cnt_v[pl.ds(g * _L, _L)]
        ivec = inv_v[pl.ds(g * _L, _L)]
        accs = list(accs)
        for k in range(_L):
            term = _class_term(g * _L + k, _bcast_lane(cvec, k),
                               _bcast_lane(ivec, k))
            accs[k % 4] = accs[k % 4] + term
        return tuple(accs)

    accs = list(_class_group)
    cvec = cnt_v[pl.ds((_C // _L) * _L, _L)]
    ivec = inv_v[pl.ds((_C // _L) * _L, _L)]
    for k in range(_C % _L):
        term = _class_term((_C // _L) * _L + k, _bcast_lane(cvec, k),
                           _bcast_lane(ivec, k))
        accs[k % 4] = accs[k % 4] + term

    res_v[...] = (accs[0] + accs[1]) + (accs[2] + accs[3])
    pltpu.sync_copy(res_v, out_hbm.at[wid])


_sc_kernel = functools.partial(
    pl.kernel,
    out_type=jax.ShapeDtypeStruct((_NW, _L), jnp.float32),
    mesh=plsc.VectorSubcoreMesh(core_axis_name="c", subcore_axis_name="s"),
    compiler_params=pltpu.CompilerParams(
        use_tc_tiling_on_sc=False, needs_layout_passes=False),
    scratch_types=[
        pltpu.VMEM((_B,), jnp.int32),
        pltpu.VMEM((2, _CHUNK, _L), jnp.float32),
        pltpu.VMEM((2, _CHUNK, _L), jnp.float32),
        pltpu.VMEM((_C, _L), jnp.float32),
        pltpu.VMEM((_C * _L,), jnp.float32),
        pltpu.VMEM((_C * _L,), jnp.float32),
        pltpu.VMEM((_CPAD,), jnp.float32),
        pltpu.VMEM((_CPAD,), jnp.float32),
        pltpu.VMEM((_L,), jnp.float32),
        pltpu.SemaphoreType.DMA,
        pltpu.SemaphoreType.DMA,
        pltpu.SemaphoreType.DMA((4,)),
    ],
)(_sc_body)


def _tc_reduce_body(x_ref, o_ref):
    o_ref[...] = jnp.sum(x_ref[...]).reshape(1, 1) * (1.0 / (_B * _D))


def kernel(modal1_inputs, modal2_inputs, targets, centers_param):
    partials = _sc_kernel(modal1_inputs, modal2_inputs, targets, centers_param)
    out = pl.pallas_call(
        _tc_reduce_body,
        out_shape=jax.ShapeDtypeStruct((1, 1), jnp.float32),
    )(partials)
    return out[0, 0]
